# Initial kernel scaffold; baseline (speedup 1.0000x reference)
import jax
import jax.numpy as jnp
from jax.experimental import pallas as pl


def kernel(x, edge_index, W1, b1, W2, b2):
    raise NotImplementedError("write your pallas kernel here")


# speedup vs baseline: 14.7766x; 14.7766x over previous
import functools

import jax
import jax.numpy as jnp
from jax import lax
from jax.experimental import pallas as pl
from jax.experimental.pallas import tpu as pltpu
from jax.experimental.pallas import tpu_sc as plsc

N = 100000
E = 1600000
NC, NS = 2, 16
NW = NC * NS
EPW = E // NW
CH = 80
NCHUNK = EPW // CH
SLICE = 6272
N2 = NS * SLICE
ZROWS = SLICE // 8
BN = 2000

_mesh = plsc.VectorSubcoreMesh(
    core_axis_name="c", subcore_axis_name="s", num_cores=NC, num_subcores=NS
)


@functools.partial(
    pl.kernel,
    out_type=jax.ShapeDtypeStruct((NC, N2), jnp.float32),
    mesh=_mesh,
    scratch_types=[
        pltpu.VMEM_SHARED((N2,), jnp.float32),
        pltpu.VMEM((CH,), jnp.int32),
        pltpu.VMEM((CH,), jnp.float32),
        pltpu.VMEM((SLICE,), jnp.float32),
    ],
)
def _sc_deg(dst_hbm, out_hbm, hist_sh, idx_v, ones_v, zbuf_v):
    cid = lax.axis_index("c")
    sid = lax.axis_index("s")
    wid = cid * NS + sid

    def fill_z(i, _):
        zbuf_v[pl.ds(i * 16, 16)] = jnp.zeros((16,), jnp.float32)
        return 0

    lax.fori_loop(0, SLICE // 16, fill_z, 0)

    def fill_o(i, _):
        ones_v[pl.ds(i * 16, 16)] = jnp.ones((16,), jnp.float32)
        return 0

    lax.fori_loop(0, CH // 16, fill_o, 0)
    pltpu.sync_copy(zbuf_v, hist_sh.at[pl.ds(sid * SLICE, SLICE)])
    plsc.subcore_barrier()

    def step(c, _):
        base = pl.multiple_of(wid * EPW + c * CH, 8)
        pltpu.sync_copy(dst_hbm.at[pl.ds(base, CH)], idx_v)
        pltpu.sync_copy(ones_v, hist_sh.at[idx_v], add=True)
        return 0

    lax.fori_loop(0, NCHUNK, step, 0)
    plsc.subcore_barrier()
    pltpu.sync_copy(
        hist_sh.at[pl.ds(sid * SLICE, SLICE)],
        out_hbm.at[cid, pl.ds(sid * SLICE, SLICE)],
    )


@functools.partial(
    pl.kernel,
    out_type=jax.ShapeDtypeStruct((NC, N2, 16), jnp.float32),
    mesh=_mesh,
    scratch_types=[
        pltpu.VMEM_SHARED((N2, 16), jnp.float32),
        pltpu.VMEM((CH,), jnp.int32),
        pltpu.VMEM((CH,), jnp.int32),
        pltpu.VMEM((CH, 16), jnp.float32),
        pltpu.VMEM((ZROWS, 16), jnp.float32),
        pltpu.SemaphoreType.DMA,
    ],
    compiler_params=pltpu.CompilerParams(use_tc_tiling_on_sc=False),
)
def _sc_agg(g_hbm, src_hbm, dst_hbm, out_hbm, acc_sh, src_v, dst_v, rows_v,
            zbuf_v, sem):
    cid = lax.axis_index("c")
    sid = lax.axis_index("s")
    wid = cid * NS + sid

    def fz(i, _):
        zbuf_v[i, :] = jnp.zeros((16,), jnp.float32)
        return 0

    lax.fori_loop(0, ZROWS, fz, 0)

    def zc(j, _):
        pltpu.sync_copy(
            zbuf_v, acc_sh.at[pl.ds(sid * SLICE + j * ZROWS, ZROWS), :]
        )
        return 0

    lax.fori_loop(0, SLICE // ZROWS, zc, 0)
    plsc.subcore_barrier()

    def step(c, _):
        base = pl.multiple_of(wid * EPW + c * CH, 8)
        pltpu.sync_copy(src_hbm.at[pl.ds(base, CH)], src_v)
        pltpu.sync_copy(dst_hbm.at[pl.ds(base, CH)], dst_v)
        pltpu.async_copy(g_hbm.at[src_v], rows_v, sem).wait()
        pltpu.sync_copy(rows_v, acc_sh.at[dst_v], add=True)
        return 0

    lax.fori_loop(0, NCHUNK, step, 0)
    plsc.subcore_barrier()
    pltpu.sync_copy(
        acc_sh.at[pl.ds(sid * SLICE, SLICE), :],
        out_hbm.at[cid, pl.ds(sid * SLICE, SLICE), :],
    )


def _tcA_body(x_ref, w_ref, degp_ref, g1_ref, dinv_ref):
    deg = degp_ref[:, 0] + degp_ref[:, 1] + 1.0
    dinv = lax.rsqrt(deg)[:, None]
    h = jnp.dot(x_ref[...], w_ref[...], preferred_element_type=jnp.float32)
    g1_ref[...] = h * dinv
    dinv_ref[...] = dinv


def _tc_A(x, W1, degp):
    return pl.pallas_call(
        _tcA_body,
        grid=(N // BN,),
        in_specs=[
            pl.BlockSpec((BN, 132), lambda i: (i, 0)),
            pl.BlockSpec((132, 16), lambda i: (0, 0)),
            pl.BlockSpec((BN, NC), lambda i: (i, 0)),
        ],
        out_specs=[
            pl.BlockSpec((BN, 16), lambda i: (i, 0)),
            pl.BlockSpec((BN, 1), lambda i: (i, 0)),
        ],
        out_shape=[
            jax.ShapeDtypeStruct((N, 16), jnp.float32),
            jax.ShapeDtypeStruct((N, 1), jnp.float32),
        ],
    )(x, W1, degp)


def _tcB_body(acc_ref, g1_ref, dinv_ref, w2p_ref, b1_ref, g2_ref):
    dinv = dinv_ref[...]
    t = (acc_ref[0] + acc_ref[1] + g1_ref[...]) * dinv + b1_ref[...]
    t = jnp.maximum(t, 0.0)
    h2 = jnp.dot(t, w2p_ref[...], preferred_element_type=jnp.float32)
    g2_ref[...] = h2 * dinv


def _tc_B(acc1, g1, dinv, w2p, b1r):
    return pl.pallas_call(
        _tcB_body,
        grid=(N // BN,),
        in_specs=[
            pl.BlockSpec((NC, BN, 16), lambda i: (0, i, 0)),
            pl.BlockSpec((BN, 16), lambda i: (i, 0)),
            pl.BlockSpec((BN, 1), lambda i: (i, 0)),
            pl.BlockSpec((16, 16), lambda i: (0, 0)),
            pl.BlockSpec((1, 16), lambda i: (0, 0)),
        ],
        out_specs=pl.BlockSpec((BN, 16), lambda i: (i, 0)),
        out_shape=jax.ShapeDtypeStruct((N, 16), jnp.float32),
    )(acc1, g1, dinv, w2p, b1r)


def _tcC_body(acc_ref, g2_ref, dinv_ref, b2_ref, out_ref):
    z16 = (acc_ref[0] + acc_ref[1] + g2_ref[...]) * dinv_ref[...]
    z = z16[:, :10] + b2_ref[...]
    m = jnp.max(z, axis=1, keepdims=True)
    zs = z - m
    out_ref[...] = zs - jnp.log(jnp.sum(jnp.exp(zs), axis=1, keepdims=True))


def _tc_C(acc2, g2, dinv, b2r):
    return pl.pallas_call(
        _tcC_body,
        grid=(N // BN,),
        in_specs=[
            pl.BlockSpec((NC, BN, 16), lambda i: (0, i, 0)),
            pl.BlockSpec((BN, 16), lambda i: (i, 0)),
            pl.BlockSpec((BN, 1), lambda i: (i, 0)),
            pl.BlockSpec((1, 10), lambda i: (0, 0)),
        ],
        out_specs=pl.BlockSpec((BN, 10), lambda i: (i, 0)),
        out_shape=jax.ShapeDtypeStruct((N, 10), jnp.float32),
    )(acc2, g2, dinv, b2r)


def kernel(x, edge_index, W1, b1, W2, b2):
    src = edge_index[0]
    dst = edge_index[1]
    degp = _sc_deg(dst)
    g1, dinv = _tc_A(x, W1, degp.T)
    acc1 = _sc_agg(g1, src, dst)
    w2p = jnp.zeros((16, 16), jnp.float32).at[:, :10].set(W2)
    g2 = _tc_B(acc1, g1, dinv, w2p, b1.reshape(1, 16))
    acc2 = _sc_agg(g2, src, dst)
    return _tc_C(acc2, g2, dinv, b2.reshape(1, 10))
